# baseline (speedup 1.0000x reference)
import functools

import jax
import jax.numpy as jnp
from jax import lax
from jax.experimental import pallas as pl
from jax.experimental.pallas import tpu as pltpu
from jax.experimental.pallas import tpu_sc as plsc

N = 10000
NPAD = 10240
F1 = 128
F2 = 64
F_OUT = 512
E = 320000
NC, NS, L = 2, 16, 16
NW = NC * NS
EPW = E // NW
K = 128
NBF = EPW // K
TAIL = EPW - NBF * K
RPT = NPAD // NS

BR = 5120
G = NPAD // BR
BRO = 5000
GO = N // BRO


def _sc_mesh():
    return plsc.VectorSubcoreMesh(
        core_axis_name="c", subcore_axis_name="s", num_cores=NC, num_subcores=NS
    )


@functools.partial(
    pl.kernel,
    out_type=(
        jax.ShapeDtypeStruct((NC, NPAD), jnp.float32),
        jax.ShapeDtypeStruct((NW, EPW), jnp.int32),
    ),
    mesh=_sc_mesh(),
    compiler_params=pltpu.CompilerParams(use_tc_tiling_on_sc=False),
    scratch_types=[
        pltpu.VMEM_SHARED((NPAD,), jnp.float32),
        pltpu.VMEM((EPW,), jnp.int32),
        pltpu.VMEM((EPW,), jnp.int32),
        pltpu.VMEM((EPW,), jnp.int32),
        pltpu.VMEM((K,), jnp.int32),
        pltpu.VMEM((K,), jnp.int32),
        pltpu.VMEM((TAIL,), jnp.int32),
        pltpu.VMEM((K,), jnp.float32),
        pltpu.VMEM((RPT,), jnp.float32),
        pltpu.SemaphoreType.DMA,
        pltpu.SemaphoreType.DMA,
    ],
)
def _deg_kernel(ei_hbm, deg_hbm, pk_hbm, dacc, ssrc, sdst, spk, db0, db1,
                dbt, ones, zbuf, sem0, sem1):
    c = lax.axis_index("c")
    s = lax.axis_index("s")
    wid = s * NC + c
    for i in range(RPT // L):
        zbuf[pl.ds(i * L, L)] = jnp.zeros((L,), jnp.float32)
    for i in range(K // L):
        ones[pl.ds(i * L, L)] = jnp.ones((L,), jnp.float32)
    pltpu.sync_copy(zbuf, dacc.at[pl.ds(s * RPT, RPT)])
    pltpu.sync_copy(ei_hbm.at[0, pl.ds(wid * EPW, EPW)], ssrc)
    pltpu.sync_copy(ei_hbm.at[1, pl.ds(wid * EPW, EPW)], sdst)

    def pack_body(i, carry):
        sv = ssrc[pl.ds(i * L, L)]
        dv = sdst[pl.ds(i * L, L)]
        spk[pl.ds(i * L, L)] = sv | (dv << 16)
        return carry

    lax.fori_loop(0, EPW // L, pack_body, 0)
    pltpu.sync_copy(spk, pk_hbm.at[wid])
    plsc.subcore_barrier()

    def fill(b, ref):
        for j in range(K // L):
            ref[pl.ds(j * L, L)] = sdst[pl.ds(b * K + j * L, L)]

    fill(0, db0)
    pltpu.make_async_copy(ones, dacc.at[db0], sem0).start(add=True)

    def body(i, carry):
        b0 = 2 * i
        fill(b0 + 1, db1)
        pltpu.make_async_copy(ones, dacc.at[db1], sem1).start(add=True)
        pltpu.make_async_copy(ones, dacc.at[db0], sem0).wait()

        @pl.when(b0 + 2 < NBF)
        def _():
            fill(b0 + 2, db0)
            pltpu.make_async_copy(ones, dacc.at[db0], sem0).start(add=True)

        pltpu.make_async_copy(ones, dacc.at[db1], sem1).wait()
        return carry

    lax.fori_loop(0, NBF // 2, body, 0)
    dbt[pl.ds(0, TAIL)] = sdst[pl.ds(NBF * K, TAIL)]
    pltpu.sync_copy(ones.at[pl.ds(0, TAIL)], dacc.at[dbt], add=True)
    plsc.subcore_barrier()
    pltpu.sync_copy(dacc.at[pl.ds(s * RPT, RPT)], deg_hbm.at[c, pl.ds(s * RPT, RPT)])


def _make_segsum64():
    F = F2
    NSLOT = 3
    assert NBF % NSLOT == 0

    @functools.partial(
        pl.kernel,
        out_type=(
            jax.ShapeDtypeStruct((NPAD, F), jnp.float32),
            jax.ShapeDtypeStruct((NPAD, F), jnp.float32),
        ),
        mesh=_sc_mesh(),
        compiler_params=pltpu.CompilerParams(use_tc_tiling_on_sc=False),
        scratch_types=(
            [pltpu.VMEM_SHARED((NPAD, F), jnp.float32),
             pltpu.VMEM((EPW,), jnp.int32)]
            + [pltpu.VMEM((K,), jnp.int32)] * 6
            + [pltpu.VMEM((TAIL,), jnp.int32)] * 2
            + [pltpu.VMEM((K, F), jnp.float32)] * 3
            + [pltpu.SemaphoreType.DMA] * 6
        ),
    )
    def seg(v_hbm, pk_hbm, outa_hbm, outb_hbm, acc, pkt,
            sb0, sb1, sb2, db0, db1, db2, sbt, dbt, r0, r1, r2,
            g0, g1, g2, s0, s1, s2):
        c = lax.axis_index("c")
        s = lax.axis_index("s")
        wid = s * NC + c
        sbs, dbs, rs, gs, ss = (sb0, sb1, sb2), (db0, db1, db2), (r0, r1, r2), (g0, g1, g2), (s0, s1, s2)

        def unpack_src(b, ref):
            for j in range(K // L):
                v = pkt[pl.ds(b * K + j * L, L)]
                ref[pl.ds(j * L, L)] = v & 0xFFFF

        def unpack_dst(b, ref):
            for j in range(K // L):
                v = pkt[pl.ds(b * K + j * L, L)]
                ref[pl.ds(j * L, L)] = lax.shift_right_logical(v, 16)

        def zbody(r, carry):
            for j in range(F // L):
                r0[r, pl.ds(j * L, L)] = jnp.zeros((L,), jnp.float32)
            return carry

        lax.fori_loop(0, K, zbody, 0)
        for i in range(RPT // K):
            pltpu.sync_copy(r0, acc.at[pl.ds(s * RPT + i * K, K)])
        pltpu.sync_copy(pk_hbm.at[wid], pkt)
        plsc.subcore_barrier()

        for k in range(NSLOT):
            unpack_src(k, sbs[k])
            pltpu.async_copy(v_hbm.at[sbs[k]], rs[k], gs[k])

        def body(i, carry):
            b = NSLOT * i
            for k in range(NSLOT):
                pltpu.make_async_copy(v_hbm.at[sbs[k]], rs[k], gs[k]).wait()
                unpack_dst(b + k, dbs[k])
                pltpu.make_async_copy(rs[k], acc.at[dbs[k]], ss[k]).start(add=True)
            for k in range(NSLOT):
                @pl.when(b + NSLOT + k < NBF)
                def _(k=k):
                    pltpu.make_async_copy(rs[k], acc.at[dbs[k]], ss[k]).wait()
                    unpack_src(b + NSLOT + k, sbs[k])
                    pltpu.async_copy(v_hbm.at[sbs[k]], rs[k], gs[k])
            return carry

        lax.fori_loop(0, NBF // NSLOT, body, 0)
        for k in range(NSLOT):
            pltpu.make_async_copy(rs[k], acc.at[dbs[k]], ss[k]).wait()
        v = pkt[pl.ds(NBF * K, TAIL)]
        sbt[pl.ds(0, TAIL)] = v & 0xFFFF
        dbt[pl.ds(0, TAIL)] = lax.shift_right_logical(v, 16)
        pltpu.async_copy(v_hbm.at[sbt], r0.at[pl.ds(0, TAIL)], g0).wait()
        pltpu.sync_copy(r0.at[pl.ds(0, TAIL)], acc.at[dbt], add=True)
        plsc.subcore_barrier()

        @pl.when(c == 0)
        def _():
            pltpu.sync_copy(acc.at[pl.ds(s * RPT, RPT)],
                            outa_hbm.at[pl.ds(s * RPT, RPT)])

        @pl.when(c == 1)
        def _():
            pltpu.sync_copy(acc.at[pl.ds(s * RPT, RPT)],
                            outb_hbm.at[pl.ds(s * RPT, RPT)])

    return seg


def _make_segsum(F):
    params = None if F == F1 else pltpu.CompilerParams(use_tc_tiling_on_sc=False)

    @functools.partial(
        pl.kernel,
        out_type=(
            jax.ShapeDtypeStruct((NPAD, F), jnp.float32),
            jax.ShapeDtypeStruct((NPAD, F), jnp.float32),
        ),
        mesh=_sc_mesh(),
        compiler_params=params,
        scratch_types=[
            pltpu.VMEM_SHARED((NPAD, F), jnp.float32),
            pltpu.VMEM((EPW,), jnp.int32),
            pltpu.VMEM((K,), jnp.int32),
            pltpu.VMEM((K,), jnp.int32),
            pltpu.VMEM((K,), jnp.int32),
            pltpu.VMEM((TAIL,), jnp.int32),
            pltpu.VMEM((TAIL,), jnp.int32),
            pltpu.VMEM((K, F), jnp.float32),
            pltpu.VMEM((K, F), jnp.float32),
            pltpu.SemaphoreType.DMA,
            pltpu.SemaphoreType.DMA,
        ],
    )
    def seg(v_hbm, pk_hbm, outa_hbm, outb_hbm, acc, pkt, sb0, sb1, db,
            sbt, dbt, rows0, rows1, sem0, sem1):
        c = lax.axis_index("c")
        s = lax.axis_index("s")
        wid = s * NC + c

        def unpack_src(b, dst_ref):
            for j in range(K // L):
                v = pkt[pl.ds(b * K + j * L, L)]
                dst_ref[pl.ds(j * L, L)] = v & 0xFFFF

        def unpack_dst(b):
            for j in range(K // L):
                v = pkt[pl.ds(b * K + j * L, L)]
                db[pl.ds(j * L, L)] = lax.shift_right_logical(v, 16)

        def zbody(r, carry):
            for j in range(F // L):
                rows0[r, pl.ds(j * L, L)] = jnp.zeros((L,), jnp.float32)
            return carry

        lax.fori_loop(0, K, zbody, 0)
        for i in range(RPT // K):
            pltpu.sync_copy(rows0, acc.at[pl.ds(s * RPT + i * K, K)])
        pltpu.sync_copy(pk_hbm.at[wid], pkt)
        plsc.subcore_barrier()

        unpack_src(0, sb0)
        pltpu.async_copy(v_hbm.at[sb0], rows0, sem0)

        def body(i, carry):
            b0 = 2 * i
            unpack_src(b0 + 1, sb1)
            pltpu.async_copy(v_hbm.at[sb1], rows1, sem1)
            pltpu.make_async_copy(v_hbm.at[sb0], rows0, sem0).wait()
            unpack_dst(b0)
            pltpu.sync_copy(rows0, acc.at[db], add=True)
            unpack_src((b0 + 2) % NBF, sb0)
            pltpu.async_copy(v_hbm.at[sb0], rows0, sem0)
            pltpu.make_async_copy(v_hbm.at[sb1], rows1, sem1).wait()
            unpack_dst(b0 + 1)
            pltpu.sync_copy(rows1, acc.at[db], add=True)
            return carry

        lax.fori_loop(0, NBF // 2, body, 0)
        pltpu.make_async_copy(v_hbm.at[sb0], rows0, sem0).wait()
        v = pkt[pl.ds(NBF * K, TAIL)]
        sbt[pl.ds(0, TAIL)] = v & 0xFFFF
        dbt[pl.ds(0, TAIL)] = lax.shift_right_logical(v, 16)
        pltpu.async_copy(v_hbm.at[sbt], rows0.at[pl.ds(0, TAIL)], sem0).wait()
        pltpu.sync_copy(rows0.at[pl.ds(0, TAIL)], acc.at[dbt], add=True)
        plsc.subcore_barrier()

        @pl.when(c == 0)
        def _():
            pltpu.sync_copy(acc.at[pl.ds(s * RPT, RPT)],
                            outa_hbm.at[pl.ds(s * RPT, RPT)])

        @pl.when(c == 1)
        def _():
            pltpu.sync_copy(acc.at[pl.ds(s * RPT, RPT)],
                            outb_hbm.at[pl.ds(s * RPT, RPT)])

    return seg


_seg128 = _make_segsum(F1)
_seg64 = _make_segsum64()


def _dinv(d0, d1):
    return lax.rsqrt(d0 + d1 + 1.0)


def _mm1_body(x_ref, w_ref, d0_ref, d1_ref, o_ref):
    dinv = _dinv(d0_ref[...], d1_ref[...])
    o_ref[...] = dinv * jnp.dot(
        x_ref[...], w_ref[...], preferred_element_type=jnp.float32
    )


def _mm2_body(s0_ref, s1_ref, t_ref, d0_ref, d1_ref, b_ref, w_ref, o_ref):
    dinv = _dinv(d0_ref[...], d1_ref[...])
    h = jnp.maximum(dinv * (s0_ref[...] + s1_ref[...] + t_ref[...]) + b_ref[...], 0.0)
    o_ref[...] = dinv * jnp.dot(h, w_ref[...], preferred_element_type=jnp.float32)


def _ew3_body(s0_ref, s1_ref, t_ref, d0_ref, d1_ref, b_ref, o_ref):
    dinv = _dinv(d0_ref[...], d1_ref[...])
    h = jnp.maximum(dinv * (s0_ref[...] + s1_ref[...] + t_ref[...]) + b_ref[...], 0.0)
    o_ref[...] = dinv * h


def _mm3_body(s0_ref, s1_ref, t_ref, d0_ref, d1_ref, w_ref, b_ref, o_ref):
    dinv = _dinv(d0_ref[...], d1_ref[...])
    agg = dinv * (s0_ref[...] + s1_ref[...] + t_ref[...])
    o_ref[...] = jnp.dot(agg, w_ref[...], preferred_element_type=jnp.float32) + b_ref[...]


def _rows_spec(F, br=BR):
    return pl.BlockSpec((br, F), lambda i: (i, 0))


def _full_spec(a, b):
    return pl.BlockSpec((a, b), lambda i: (0, 0))


def _mm1(x, w1, d0, d1):
    return pl.pallas_call(
        _mm1_body,
        grid=(G,),
        in_specs=[_rows_spec(F1), _full_spec(F1, F1), _rows_spec(1), _rows_spec(1)],
        out_specs=_rows_spec(F1),
        out_shape=jax.ShapeDtypeStruct((NPAD, F1), jnp.float32),
    )(x, w1, d0, d1)


def _mm2(s0, s1, t1, d0, d1, b1, w2):
    return pl.pallas_call(
        _mm2_body,
        grid=(G,),
        in_specs=[
            _rows_spec(F1), _rows_spec(F1), _rows_spec(F1),
            _rows_spec(1), _rows_spec(1), _full_spec(1, F1), _full_spec(F1, F2),
        ],
        out_specs=_rows_spec(F2),
        out_shape=jax.ShapeDtypeStruct((NPAD, F2), jnp.float32),
    )(s0, s1, t1, d0, d1, b1, w2)


def _ew3(s0, s1, t2, d0, d1, b2):
    return pl.pallas_call(
        _ew3_body,
        grid=(G,),
        in_specs=[
            _rows_spec(F2), _rows_spec(F2), _rows_spec(F2),
            _rows_spec(1), _rows_spec(1), _full_spec(1, F2),
        ],
        out_specs=_rows_spec(F2),
        out_shape=jax.ShapeDtypeStruct((NPAD, F2), jnp.float32),
    )(s0, s1, t2, d0, d1, b2)


def _mm3(s0, s1, t3, d0, d1, w3, b3):
    return pl.pallas_call(
        _mm3_body,
        grid=(GO,),
        in_specs=[
            _rows_spec(F2, BRO), _rows_spec(F2, BRO), _rows_spec(F2, BRO),
            _rows_spec(1, BRO), _rows_spec(1, BRO),
            _full_spec(F2, F_OUT), _full_spec(1, F_OUT),
        ],
        out_specs=_rows_spec(F_OUT, BRO),
        out_shape=jax.ShapeDtypeStruct((N, F_OUT), jnp.float32),
    )(s0, s1, t3, d0, d1, w3, b3)


def kernel(x, edge_index, W1, b1, W2, b2, W3, b3):
    ei = jnp.asarray(edge_index, jnp.int32)
    degp, pk = _deg_kernel(ei)
    d0 = degp[0].reshape(NPAD, 1)
    d1 = degp[1].reshape(NPAD, 1)
    t1 = _mm1(x, W1, d0, d1)
    s1a, s1b = _seg128(t1, pk)
    t2 = _mm2(s1a, s1b, t1, d0, d1, b1.reshape(1, F1), W2)
    s2a, s2b = _seg64(t2, pk)
    t3 = _ew3(s2a, s2b, t2, d0, d1, b2.reshape(1, F2))
    s3a, s3b = _seg64(t3, pk)
    return _mm3(s3a, s3b, t3, d0, d1, W3, b3.reshape(1, F_OUT))


# speedup vs baseline: 1.0044x; 1.0044x over previous
import functools

import jax
import jax.numpy as jnp
from jax import lax
from jax.experimental import pallas as pl
from jax.experimental.pallas import tpu as pltpu
from jax.experimental.pallas import tpu_sc as plsc

N = 10000
NPAD = 10240
F1 = 128
F2 = 64
F_OUT = 512
E = 320000
NC, NS, L = 2, 16, 16
NW = NC * NS
EPW = E // NW
K = 128
NBF = EPW // K
TAIL = EPW - NBF * K
RPT = NPAD // NS

BR = 5120
G = NPAD // BR
BRO = 5000
GO = N // BRO


def _sc_mesh():
    return plsc.VectorSubcoreMesh(
        core_axis_name="c", subcore_axis_name="s", num_cores=NC, num_subcores=NS
    )


@functools.partial(
    pl.kernel,
    out_type=(
        jax.ShapeDtypeStruct((NC, NPAD), jnp.float32),
        jax.ShapeDtypeStruct((NW, EPW), jnp.int32),
    ),
    mesh=_sc_mesh(),
    scratch_types=[
        pltpu.VMEM_SHARED((NPAD,), jnp.float32),
        pltpu.VMEM((EPW,), jnp.int32),
        pltpu.VMEM((EPW,), jnp.int32),
        pltpu.VMEM((EPW,), jnp.int32),
        pltpu.VMEM((K,), jnp.int32),
        pltpu.VMEM((K,), jnp.int32),
        pltpu.VMEM((TAIL,), jnp.int32),
        pltpu.VMEM((K,), jnp.float32),
        pltpu.VMEM((RPT,), jnp.float32),
        pltpu.SemaphoreType.DMA,
        pltpu.SemaphoreType.DMA,
    ],
)
def _deg_kernel(ei_hbm, deg_hbm, pk_hbm, dacc, ssrc, sdst, spk, db0, db1,
                dbt, ones, zbuf, sem0, sem1):
    c = lax.axis_index("c")
    s = lax.axis_index("s")
    wid = s * NC + c
    for i in range(RPT // L):
        zbuf[pl.ds(i * L, L)] = jnp.zeros((L,), jnp.float32)
    for i in range(K // L):
        ones[pl.ds(i * L, L)] = jnp.ones((L,), jnp.float32)
    pltpu.sync_copy(zbuf, dacc.at[pl.ds(s * RPT, RPT)])
    pltpu.sync_copy(ei_hbm.at[0, wid], ssrc)
    pltpu.sync_copy(ei_hbm.at[1, wid], sdst)

    def pack_body(i, carry):
        sv = ssrc[pl.ds(i * L, L)]
        dv = sdst[pl.ds(i * L, L)]
        spk[pl.ds(i * L, L)] = sv | (dv << 16)
        return carry

    lax.fori_loop(0, EPW // L, pack_body, 0)
    pltpu.sync_copy(spk, pk_hbm.at[wid])
    plsc.subcore_barrier()

    def fill(b, ref):
        for j in range(K // L):
            ref[pl.ds(j * L, L)] = sdst[pl.ds(b * K + j * L, L)]

    fill(0, db0)
    pltpu.make_async_copy(ones, dacc.at[db0], sem0).start(add=True)

    def body(i, carry):
        b0 = 2 * i
        fill(b0 + 1, db1)
        pltpu.make_async_copy(ones, dacc.at[db1], sem1).start(add=True)
        pltpu.make_async_copy(ones, dacc.at[db0], sem0).wait()

        @pl.when(b0 + 2 < NBF)
        def _():
            fill(b0 + 2, db0)
            pltpu.make_async_copy(ones, dacc.at[db0], sem0).start(add=True)

        pltpu.make_async_copy(ones, dacc.at[db1], sem1).wait()
        return carry

    lax.fori_loop(0, NBF // 2, body, 0)
    dbt[pl.ds(0, TAIL)] = sdst[pl.ds(NBF * K, TAIL)]
    pltpu.sync_copy(ones.at[pl.ds(0, TAIL)], dacc.at[dbt], add=True)
    plsc.subcore_barrier()
    pltpu.sync_copy(dacc.at[pl.ds(s * RPT, RPT)], deg_hbm.at[c, pl.ds(s * RPT, RPT)])


def _make_segsum64():
    F = F2
    NSLOT = 3
    assert NBF % NSLOT == 0

    @functools.partial(
        pl.kernel,
        out_type=(
            jax.ShapeDtypeStruct((NPAD, F), jnp.float32),
            jax.ShapeDtypeStruct((NPAD, F), jnp.float32),
        ),
        mesh=_sc_mesh(),
        compiler_params=pltpu.CompilerParams(use_tc_tiling_on_sc=False),
        scratch_types=(
            [pltpu.VMEM_SHARED((NPAD, F), jnp.float32),
             pltpu.VMEM((EPW,), jnp.int32)]
            + [pltpu.VMEM((K,), jnp.int32)] * 6
            + [pltpu.VMEM((TAIL,), jnp.int32)] * 2
            + [pltpu.VMEM((K, F), jnp.float32)] * 3
            + [pltpu.SemaphoreType.DMA] * 6
        ),
    )
    def seg(v_hbm, pk_hbm, outa_hbm, outb_hbm, acc, pkt,
            sb0, sb1, sb2, db0, db1, db2, sbt, dbt, r0, r1, r2,
            g0, g1, g2, s0, s1, s2):
        c = lax.axis_index("c")
        s = lax.axis_index("s")
        wid = s * NC + c
        sbs, dbs, rs, gs, ss = (sb0, sb1, sb2), (db0, db1, db2), (r0, r1, r2), (g0, g1, g2), (s0, s1, s2)

        def unpack_src(b, ref):
            for j in range(K // L):
                v = pkt[pl.ds(b * K + j * L, L)]
                ref[pl.ds(j * L, L)] = v & 0xFFFF

        def unpack_dst(b, ref):
            for j in range(K // L):
                v = pkt[pl.ds(b * K + j * L, L)]
                ref[pl.ds(j * L, L)] = lax.shift_right_logical(v, 16)

        def zbody(r, carry):
            for j in range(F // L):
                r0[r, pl.ds(j * L, L)] = jnp.zeros((L,), jnp.float32)
            return carry

        lax.fori_loop(0, K, zbody, 0)
        for i in range(RPT // K):
            pltpu.sync_copy(r0, acc.at[pl.ds(s * RPT + i * K, K)])
        pltpu.sync_copy(pk_hbm.at[wid], pkt)
        plsc.subcore_barrier()

        for k in range(NSLOT):
            unpack_src(k, sbs[k])
            pltpu.async_copy(v_hbm.at[sbs[k]], rs[k], gs[k])

        def body(i, carry):
            b = NSLOT * i
            for k in range(NSLOT):
                pltpu.make_async_copy(v_hbm.at[sbs[k]], rs[k], gs[k]).wait()
                unpack_dst(b + k, dbs[k])
                pltpu.make_async_copy(rs[k], acc.at[dbs[k]], ss[k]).start(add=True)
            for k in range(NSLOT):
                @pl.when(b + NSLOT + k < NBF)
                def _(k=k):
                    pltpu.make_async_copy(rs[k], acc.at[dbs[k]], ss[k]).wait()
                    unpack_src(b + NSLOT + k, sbs[k])
                    pltpu.async_copy(v_hbm.at[sbs[k]], rs[k], gs[k])
            return carry

        lax.fori_loop(0, NBF // NSLOT, body, 0)
        for k in range(NSLOT):
            pltpu.make_async_copy(rs[k], acc.at[dbs[k]], ss[k]).wait()
        v = pkt[pl.ds(NBF * K, TAIL)]
        sbt[pl.ds(0, TAIL)] = v & 0xFFFF
        dbt[pl.ds(0, TAIL)] = lax.shift_right_logical(v, 16)
        pltpu.async_copy(v_hbm.at[sbt], r0.at[pl.ds(0, TAIL)], g0).wait()
        pltpu.sync_copy(r0.at[pl.ds(0, TAIL)], acc.at[dbt], add=True)
        plsc.subcore_barrier()

        @pl.when(c == 0)
        def _():
            pltpu.sync_copy(acc.at[pl.ds(s * RPT, RPT)],
                            outa_hbm.at[pl.ds(s * RPT, RPT)])

        @pl.when(c == 1)
        def _():
            pltpu.sync_copy(acc.at[pl.ds(s * RPT, RPT)],
                            outb_hbm.at[pl.ds(s * RPT, RPT)])

    return seg


def _make_segsum(F):
    params = None if F == F1 else pltpu.CompilerParams(use_tc_tiling_on_sc=False)

    @functools.partial(
        pl.kernel,
        out_type=(
            jax.ShapeDtypeStruct((NPAD, F), jnp.float32),
            jax.ShapeDtypeStruct((NPAD, F), jnp.float32),
        ),
        mesh=_sc_mesh(),
        compiler_params=params,
        scratch_types=[
            pltpu.VMEM_SHARED((NPAD, F), jnp.float32),
            pltpu.VMEM((EPW,), jnp.int32),
            pltpu.VMEM((K,), jnp.int32),
            pltpu.VMEM((K,), jnp.int32),
            pltpu.VMEM((K,), jnp.int32),
            pltpu.VMEM((TAIL,), jnp.int32),
            pltpu.VMEM((TAIL,), jnp.int32),
            pltpu.VMEM((K, F), jnp.float32),
            pltpu.VMEM((K, F), jnp.float32),
            pltpu.SemaphoreType.DMA,
            pltpu.SemaphoreType.DMA,
        ],
    )
    def seg(v_hbm, pk_hbm, outa_hbm, outb_hbm, acc, pkt, sb0, sb1, db,
            sbt, dbt, rows0, rows1, sem0, sem1):
        c = lax.axis_index("c")
        s = lax.axis_index("s")
        wid = s * NC + c

        def unpack_src(b, dst_ref):
            for j in range(K // L):
                v = pkt[pl.ds(b * K + j * L, L)]
                dst_ref[pl.ds(j * L, L)] = v & 0xFFFF

        def unpack_dst(b):
            for j in range(K // L):
                v = pkt[pl.ds(b * K + j * L, L)]
                db[pl.ds(j * L, L)] = lax.shift_right_logical(v, 16)

        def zbody(r, carry):
            for j in range(F // L):
                rows0[r, pl.ds(j * L, L)] = jnp.zeros((L,), jnp.float32)
            return carry

        lax.fori_loop(0, K, zbody, 0)
        for i in range(RPT // K):
            pltpu.sync_copy(rows0, acc.at[pl.ds(s * RPT + i * K, K)])
        pltpu.sync_copy(pk_hbm.at[wid], pkt)
        plsc.subcore_barrier()

        unpack_src(0, sb0)
        pltpu.async_copy(v_hbm.at[sb0], rows0, sem0)

        def body(i, carry):
            b0 = 2 * i
            unpack_src(b0 + 1, sb1)
            pltpu.async_copy(v_hbm.at[sb1], rows1, sem1)
            pltpu.make_async_copy(v_hbm.at[sb0], rows0, sem0).wait()
            unpack_dst(b0)
            pltpu.sync_copy(rows0, acc.at[db], add=True)
            unpack_src((b0 + 2) % NBF, sb0)
            pltpu.async_copy(v_hbm.at[sb0], rows0, sem0)
            pltpu.make_async_copy(v_hbm.at[sb1], rows1, sem1).wait()
            unpack_dst(b0 + 1)
            pltpu.sync_copy(rows1, acc.at[db], add=True)
            return carry

        lax.fori_loop(0, NBF // 2, body, 0)
        pltpu.make_async_copy(v_hbm.at[sb0], rows0, sem0).wait()
        v = pkt[pl.ds(NBF * K, TAIL)]
        sbt[pl.ds(0, TAIL)] = v & 0xFFFF
        dbt[pl.ds(0, TAIL)] = lax.shift_right_logical(v, 16)
        pltpu.async_copy(v_hbm.at[sbt], rows0.at[pl.ds(0, TAIL)], sem0).wait()
        pltpu.sync_copy(rows0.at[pl.ds(0, TAIL)], acc.at[dbt], add=True)
        plsc.subcore_barrier()

        @pl.when(c == 0)
        def _():
            pltpu.sync_copy(acc.at[pl.ds(s * RPT, RPT)],
                            outa_hbm.at[pl.ds(s * RPT, RPT)])

        @pl.when(c == 1)
        def _():
            pltpu.sync_copy(acc.at[pl.ds(s * RPT, RPT)],
                            outb_hbm.at[pl.ds(s * RPT, RPT)])

    return seg


_seg128 = _make_segsum(F1)
_seg64 = _make_segsum64()


def _dinv(d0, d1):
    return lax.rsqrt(d0 + d1 + 1.0)


def _mm1_body(x_ref, w_ref, d0_ref, d1_ref, o_ref):
    dinv = _dinv(d0_ref[...], d1_ref[...])
    o_ref[...] = dinv * jnp.dot(
        x_ref[...], w_ref[...], preferred_element_type=jnp.float32
    )


def _mm2_body(s0_ref, s1_ref, t_ref, d0_ref, d1_ref, b_ref, w_ref, o_ref):
    dinv = _dinv(d0_ref[...], d1_ref[...])
    h = jnp.maximum(dinv * (s0_ref[...] + s1_ref[...] + t_ref[...]) + b_ref[...], 0.0)
    o_ref[...] = dinv * jnp.dot(h, w_ref[...], preferred_element_type=jnp.float32)


def _ew3_body(s0_ref, s1_ref, t_ref, d0_ref, d1_ref, b_ref, o_ref):
    dinv = _dinv(d0_ref[...], d1_ref[...])
    h = jnp.maximum(dinv * (s0_ref[...] + s1_ref[...] + t_ref[...]) + b_ref[...], 0.0)
    o_ref[...] = dinv * h


def _mm3_body(s0_ref, s1_ref, t_ref, d0_ref, d1_ref, w_ref, b_ref, o_ref):
    dinv = _dinv(d0_ref[...], d1_ref[...])
    agg = dinv * (s0_ref[...] + s1_ref[...] + t_ref[...])
    o_ref[...] = jnp.dot(agg, w_ref[...], preferred_element_type=jnp.float32) + b_ref[...]


def _rows_spec(F, br=BR):
    return pl.BlockSpec((br, F), lambda i: (i, 0))


def _full_spec(a, b):
    return pl.BlockSpec((a, b), lambda i: (0, 0))


def _mm1(x, w1, d0, d1):
    return pl.pallas_call(
        _mm1_body,
        grid=(G,),
        in_specs=[_rows_spec(F1), _full_spec(F1, F1), _rows_spec(1), _rows_spec(1)],
        out_specs=_rows_spec(F1),
        out_shape=jax.ShapeDtypeStruct((NPAD, F1), jnp.float32),
    )(x, w1, d0, d1)


def _mm2(s0, s1, t1, d0, d1, b1, w2):
    return pl.pallas_call(
        _mm2_body,
        grid=(G,),
        in_specs=[
            _rows_spec(F1), _rows_spec(F1), _rows_spec(F1),
            _rows_spec(1), _rows_spec(1), _full_spec(1, F1), _full_spec(F1, F2),
        ],
        out_specs=_rows_spec(F2),
        out_shape=jax.ShapeDtypeStruct((NPAD, F2), jnp.float32),
    )(s0, s1, t1, d0, d1, b1, w2)


def _ew3(s0, s1, t2, d0, d1, b2):
    return pl.pallas_call(
        _ew3_body,
        grid=(G,),
        in_specs=[
            _rows_spec(F2), _rows_spec(F2), _rows_spec(F2),
            _rows_spec(1), _rows_spec(1), _full_spec(1, F2),
        ],
        out_specs=_rows_spec(F2),
        out_shape=jax.ShapeDtypeStruct((NPAD, F2), jnp.float32),
    )(s0, s1, t2, d0, d1, b2)


def _mm3(s0, s1, t3, d0, d1, w3, b3):
    return pl.pallas_call(
        _mm3_body,
        grid=(GO,),
        in_specs=[
            _rows_spec(F2, BRO), _rows_spec(F2, BRO), _rows_spec(F2, BRO),
            _rows_spec(1, BRO), _rows_spec(1, BRO),
            _full_spec(F2, F_OUT), _full_spec(1, F_OUT),
        ],
        out_specs=_rows_spec(F_OUT, BRO),
        out_shape=jax.ShapeDtypeStruct((N, F_OUT), jnp.float32),
    )(s0, s1, t3, d0, d1, w3, b3)


def kernel(x, edge_index, W1, b1, W2, b2, W3, b3):
    ei = jnp.asarray(edge_index, jnp.int32).reshape(2, NW, EPW)
    degp, pk = _deg_kernel(ei)
    d0 = degp[0].reshape(NPAD, 1)
    d1 = degp[1].reshape(NPAD, 1)
    t1 = _mm1(x, W1, d0, d1)
    s1a, s1b = _seg128(t1, pk)
    t2 = _mm2(s1a, s1b, t1, d0, d1, b1.reshape(1, F1), W2)
    s2a, s2b = _seg64(t2, pk)
    t3 = _ew3(s2a, s2b, t2, d0, d1, b2.reshape(1, F2))
    s3a, s3b = _seg64(t3, pk)
    return _mm3(s3a, s3b, t3, d0, d1, W3, b3.reshape(1, F_OUT))
